# Initial kernel scaffold; baseline (speedup 1.0000x reference)
import jax
import jax.numpy as jnp
from jax.experimental import pallas as pl


def kernel(x, edge_index, Wq, bq, Wk, bk, Wv, bv, Ws, bs):
    raise NotImplementedError("write your pallas kernel here")


# speedup vs baseline: 4.7929x; 4.7929x over previous
import functools

import numpy as np

import jax
import jax.numpy as jnp
from jax import lax
from jax.experimental import pallas as pl
from jax.experimental.pallas import tpu as pltpu
from jax.experimental.pallas import tpu_sc as plsc

N = 10000
E = 320000
D = 128
NC = 2
NS = 16
NW = NC * NS
NPAD = 10240
NT = NPAD // NW
SCCH = 3200
CH = 40
PCAP = 14400
PBUF = PCAP + 2 * CH + 16
TRASH = PBUF - 16
INV_SQRT = 1.0 / (128.0 ** 0.5)


BN = 400


def _proj_body(x_ref, wt_ref, b_ref, o_ref):
    o_ref[...] = (
        jnp.dot(x_ref[...], wt_ref[...], preferred_element_type=jnp.float32)
        + b_ref[...]
    )


def _proj(x, wt, b):
    return pl.pallas_call(
        _proj_body,
        grid=(N // BN,),
        in_specs=[
            pl.BlockSpec((BN, 128), lambda i: (i, 0)),
            pl.BlockSpec((128, 512), lambda i: (0, 0)),
            pl.BlockSpec((1, 512), lambda i: (0, 0)),
        ],
        out_specs=pl.BlockSpec((BN, 512), lambda i: (i, 0)),
        out_shape=jax.ShapeDtypeStruct((N, 512), jnp.float32),
    )(x, wt, b)


def _comb_body(p_ref, d_ref, s_ref, o_ref):
    den = d_ref[:, 0:1]
    o_ref[...] = p_ref[...] / (den + 1e-16) + s_ref[...]


def _combine(p, d, skip):
    return pl.pallas_call(
        _comb_body,
        grid=(N // BN,),
        in_specs=[
            pl.BlockSpec((BN, 128), lambda i: (i, 0)),
            pl.BlockSpec((BN, 16), lambda i: (i, 0)),
            pl.BlockSpec((BN, 128), lambda i: (i, 0)),
        ],
        out_specs=pl.BlockSpec((BN, 128), lambda i: (i, 0)),
        out_shape=jax.ShapeDtypeStruct((N, 128), jnp.float32),
    )(p, d, skip)


def _permute16(x, idx):
    return lax.gather(
        x,
        idx[:, None],
        dimension_numbers=lax.GatherDimensionNumbers(
            offset_dims=(), collapsed_slice_dims=(0,), start_index_map=(0,)
        ),
        slice_sizes=(1,),
        mode=lax.GatherScatterMode.PROMISE_IN_BOUNDS,
    )


def _sc_edge(q, k, v, src, dst):
    mesh = plsc.VectorSubcoreMesh(core_axis_name="c", subcore_axis_name="s")

    @functools.partial(
        pl.kernel,
        mesh=mesh,
        out_type=[
            jax.ShapeDtypeStruct((NPAD, D), jnp.float32),
            jax.ShapeDtypeStruct((NPAD, 16), jnp.float32),
        ],
        scratch_types=[
            pltpu.VMEM((SCCH,), jnp.int32),
            pltpu.VMEM((SCCH,), jnp.int32),
            pltpu.VMEM((PBUF,), jnp.int32),
            pltpu.VMEM((PBUF,), jnp.int32),
            pltpu.VMEM((PBUF,), jnp.int32),
            pltpu.VMEM((16,), jnp.int32),
            pltpu.VMEM((CH, D), jnp.float32),
            pltpu.VMEM((CH, D), jnp.float32),
            pltpu.VMEM((CH, D), jnp.float32),
            pltpu.VMEM((NT, D), jnp.float32),
            pltpu.VMEM((NT, 16), jnp.float32),
            pltpu.SemaphoreType.DMA,
        ],
        compiler_params=pltpu.CompilerParams(use_tc_tiling_on_sc=False),
    )
    def body(q_hbm, k_hbm, v_hbm, src_hbm, dst_hbm,
             outp_hbm, denp_hbm,
             sidx, didx, psrc, pdst, ppack, bounce, qv, kv, vv, accv, denv,
             sem):
        c = lax.axis_index("c")
        s = lax.axis_index("s")
        wid = c * NS + s
        lo = wid * NT

        lane = lax.iota(jnp.int32, 16)
        zero = jnp.zeros((16,), jnp.float32)
        perms = [jnp.bitwise_xor(lane, jnp.int32(sh)) for sh in (8, 4, 2, 1)]
        shift_perms = [jnp.bitwise_and(lane - sh, 15) for sh in (1, 2, 4, 8)]
        shift_gates = [jnp.where(lane >= sh, 1, 0) for sh in (1, 2, 4, 8)]
        jplus1 = lane + 1

        def zfill_body(i, carry):
            for j in range(8):
                accv[i, pl.ds(16 * j, 16)] = zero
            denv[i, :] = zero
            return carry

        lax.fori_loop(0, NT, zfill_body, 0)

        def scan_body(g, cnt):
            base = g * SCCH
            pltpu.sync_copy(src_hbm.at[pl.ds(base, SCCH)], sidx)
            pltpu.sync_copy(dst_hbm.at[pl.ds(base, SCCH)], didx)

            def vec_body(i, cnt2):
                dv = didx[pl.ds(16 * i, 16)]
                sv = sidx[pl.ds(16 * i, 16)]
                m = (dv >= lo) & (dv < lo + NT)
                mi = jnp.where(m, 1, 0)
                incl = mi
                for sp, sg in zip(shift_perms, shift_gates):
                    incl = incl + _permute16(incl, sp) * sg
                bounce[:] = incl
                count = bounce[:][15]
                pos = jnp.zeros((16,), jnp.int32)
                for sh in (8, 4, 2, 1):
                    probe = _permute16(incl, pos + (sh - 1))
                    pos = pos + jnp.where(probe < jplus1, sh, 0)
                vals = dv * 32768 + sv
                svals = _permute16(vals, pos)
                off = jnp.minimum(cnt2, PCAP)
                ppack[pl.ds(off, 16)] = svals
                return off + count

            return lax.fori_loop(0, SCCH // 16, vec_body, cnt)

        cnt = lax.fori_loop(0, E // SCCH, scan_body, 0)

        lovec = jnp.full((16,), lo * 32768, dtype=jnp.int32)
        for t in range(CH // 16 + 1):
            ppack[pl.ds(cnt + 16 * t, 16)] = lovec
        nchunks = (cnt + CH - 1) // CH

        def unpack_body(j, carry):
            val = ppack[pl.ds(16 * j, 16)]
            pdst[pl.ds(16 * j, 16)] = jnp.right_shift(val, 15)
            psrc[pl.ds(16 * j, 16)] = jnp.bitwise_and(val, 32767)
            return carry

        lax.fori_loop(0, (cnt + CH + 15) // 16, unpack_body, 0)

        def chunk_body(g, carry):
            base = g * CH
            d1 = pltpu.async_copy(q_hbm.at[pdst.at[pl.ds(base, CH)]], qv, sem)
            d2 = pltpu.async_copy(k_hbm.at[psrc.at[pl.ds(base, CH)]], kv, sem)
            d3 = pltpu.async_copy(v_hbm.at[psrc.at[pl.ds(base, CH)]], vv, sem)
            d1.wait()
            d2.wait()
            d3.wait()

            def edge_body(i, carry2):
                acc = qv[i, pl.ds(0, 16)] * kv[i, pl.ds(0, 16)]
                for j in range(1, 8):
                    acc = acc + qv[i, pl.ds(16 * j, 16)] * kv[i, pl.ds(16 * j, 16)]
                for p in perms:
                    acc = acc + _permute16(acc, p)
                exvec = jnp.exp(acc * INV_SQRT)
                isreal = (base + i) < cnt
                exvec = jnp.where(isreal, exvec, 0.0)
                dstloc = pdst[pl.ds(base + i, 16)][0] - lo
                for j in range(8):
                    contrib = vv[i, pl.ds(16 * j, 16)] * exvec
                    plsc.addupdate(accv.at[dstloc, pl.ds(16 * j, 16)], contrib)
                exrow = jnp.where(lane == 0, exvec, 0.0)
                plsc.addupdate(denv.at[dstloc], exrow)
                return carry2

            lax.fori_loop(0, CH, edge_body, 0)
            return carry

        lax.fori_loop(0, nchunks, chunk_body, 0)

        pltpu.sync_copy(accv, outp_hbm.at[pl.ds(lo, NT)])
        pltpu.sync_copy(denv, denp_hbm.at[pl.ds(lo, NT)])

    return body(q, k, v, src, dst)


def kernel(x, edge_index, Wq, bq, Wk, bk, Wv, bv, Ws, bs):
    wt = jnp.concatenate([Wq.T, Wk.T, Wv.T, Ws.T], axis=1)
    b = jnp.concatenate([bq, bk, bv, bs]).reshape(1, 512)
    proj = _proj(x, wt, b)
    q = proj[:, 0:128]
    k = proj[:, 128:256]
    v = proj[:, 256:384]
    skip = proj[:, 384:512]
    src = edge_index[0].astype(jnp.int32)
    dst = edge_index[1].astype(jnp.int32)
    outp, denp = _sc_edge(q, k, v, src, dst)
    return _combine(outp[:N], denp[:N], skip)


# speedup vs baseline: 5.4401x; 1.1350x over previous
import functools

import numpy as np

import jax
import jax.numpy as jnp
from jax import lax
from jax.experimental import pallas as pl
from jax.experimental.pallas import tpu as pltpu
from jax.experimental.pallas import tpu_sc as plsc

N = 10000
E = 320000
D = 128
NC = 2
NS = 16
NW = NC * NS
NPAD = 10240
NT = NPAD // NW
SCCH = 3200
CH = 40
PCAP = 12992
PBUF = PCAP + 4 * CH + 16
INV_SQRT = 1.0 / (128.0 ** 0.5)


BN = 400


def _proj_body(x_ref, wt_ref, b_ref, o_ref):
    o_ref[...] = (
        jnp.dot(x_ref[...], wt_ref[...], preferred_element_type=jnp.float32)
        + b_ref[...]
    )


def _proj(x, wt, b):
    return pl.pallas_call(
        _proj_body,
        grid=(N // BN,),
        in_specs=[
            pl.BlockSpec((BN, 128), lambda i: (i, 0)),
            pl.BlockSpec((128, 512), lambda i: (0, 0)),
            pl.BlockSpec((1, 512), lambda i: (0, 0)),
        ],
        out_specs=pl.BlockSpec((BN, 512), lambda i: (i, 0)),
        out_shape=jax.ShapeDtypeStruct((N, 512), jnp.float32),
    )(x, wt, b)


def _comb_body(p_ref, d_ref, s_ref, o_ref):
    den = d_ref[:, 0:1]
    o_ref[...] = p_ref[...] / (den + 1e-16) + s_ref[...]


def _combine(p, d, skip):
    return pl.pallas_call(
        _comb_body,
        grid=(N // BN,),
        in_specs=[
            pl.BlockSpec((BN, 128), lambda i: (i, 0)),
            pl.BlockSpec((BN, 16), lambda i: (i, 0)),
            pl.BlockSpec((BN, 128), lambda i: (i, 0)),
        ],
        out_specs=pl.BlockSpec((BN, 128), lambda i: (i, 0)),
        out_shape=jax.ShapeDtypeStruct((N, 128), jnp.float32),
    )(p, d, skip)


def _permute16(x, idx):
    return lax.gather(
        x,
        idx[:, None],
        dimension_numbers=lax.GatherDimensionNumbers(
            offset_dims=(), collapsed_slice_dims=(0,), start_index_map=(0,)
        ),
        slice_sizes=(1,),
        mode=lax.GatherScatterMode.PROMISE_IN_BOUNDS,
    )


def _sc_edge(q, k, v, src, dst):
    mesh = plsc.VectorSubcoreMesh(core_axis_name="c", subcore_axis_name="s")

    @functools.partial(
        pl.kernel,
        mesh=mesh,
        out_type=[
            jax.ShapeDtypeStruct((NPAD, D), jnp.float32),
            jax.ShapeDtypeStruct((NPAD, 16), jnp.float32),
        ],
        scratch_types=[
            pltpu.VMEM((2, SCCH), jnp.int32),
            pltpu.VMEM((2, SCCH), jnp.int32),
            pltpu.VMEM((PBUF,), jnp.int32),
            pltpu.VMEM((PBUF,), jnp.int32),
            pltpu.VMEM((PBUF,), jnp.int32),
            pltpu.VMEM((16,), jnp.int32),
            pltpu.VMEM((2, CH, D), jnp.float32),
            pltpu.VMEM((2, CH, D), jnp.float32),
            pltpu.VMEM((2, CH, D), jnp.float32),
            pltpu.VMEM((NT, D), jnp.float32),
            pltpu.VMEM((NT, 16), jnp.float32),
            pltpu.SemaphoreType.DMA,
            pltpu.SemaphoreType.DMA,
            pltpu.SemaphoreType.DMA,
            pltpu.SemaphoreType.DMA,
        ],
        compiler_params=pltpu.CompilerParams(use_tc_tiling_on_sc=False),
    )
    def body(q_hbm, k_hbm, v_hbm, src_hbm, dst_hbm,
             outp_hbm, denp_hbm,
             sidx, didx, psrc, pdst, ppack, bounce, qv, kv, vv, accv, denv,
             semA, semB, sem0, sem1):
        c = lax.axis_index("c")
        s = lax.axis_index("s")
        wid = c * NS + s
        lo = wid * NT

        lane = lax.iota(jnp.int32, 16)
        zero = jnp.zeros((16,), jnp.float32)
        perms = [jnp.bitwise_xor(lane, jnp.int32(sh)) for sh in (8, 4, 2, 1)]
        shift_perms = [jnp.bitwise_and(lane - sh, 15) for sh in (1, 2, 4, 8)]
        shift_gates = [jnp.where(lane >= sh, 1, 0) for sh in (1, 2, 4, 8)]
        jplus1 = lane + 1

        def zfill_body(i, carry):
            for j in range(8):
                accv[i, pl.ds(16 * j, 16)] = zero
            denv[i, :] = zero
            return carry

        lax.fori_loop(0, NT, zfill_body, 0)

        NSCAN = E // SCCH

        def issue_idx(g, b, sm):
            base = g * SCCH
            pltpu.async_copy(src_hbm.at[pl.ds(base, SCCH)], sidx.at[b], sm)
            pltpu.async_copy(dst_hbm.at[pl.ds(base, SCCH)], didx.at[b], sm)

        def wait_idx(b, sm):
            pltpu.make_async_copy(src_hbm.at[pl.ds(0, SCCH)], sidx.at[b], sm).wait()
            pltpu.make_async_copy(dst_hbm.at[pl.ds(0, SCCH)], didx.at[b], sm).wait()

        def scan_chunk(b, cnt):
            def vec_body(i, cnt2):
                dv = didx[b, pl.ds(16 * i, 16)]
                sv = sidx[b, pl.ds(16 * i, 16)]
                m = (dv >= lo) & (dv < lo + NT)
                mi = jnp.where(m, 1, 0)
                incl = mi
                for sp, sg in zip(shift_perms, shift_gates):
                    incl = incl + _permute16(incl, sp) * sg
                bounce[:] = incl
                count = bounce[:][15]

                def store_path(cnt3):
                    pos = jnp.zeros((16,), jnp.int32)
                    for sh in (8, 4, 2, 1):
                        probe = _permute16(incl, pos + (sh - 1))
                        pos = pos + jnp.where(probe < jplus1, sh, 0)
                    vals = dv * 32768 + sv
                    svals = _permute16(vals, pos)
                    off = jnp.minimum(cnt3, PCAP)
                    ppack[pl.ds(off, 16)] = svals
                    return off + count

                return lax.cond(count > 0, store_path, lambda x: x, cnt2)

            return lax.fori_loop(0, SCCH // 16, vec_body, cnt)

        issue_idx(0, 0, semA)

        def scan_pair(g2, cnt):
            g0 = 2 * g2
            issue_idx(g0 + 1, 1, semB)
            wait_idx(0, semA)
            cnt = scan_chunk(0, cnt)

            @pl.when(g0 + 2 < NSCAN)
            def _():
                issue_idx(g0 + 2, 0, semA)

            wait_idx(1, semB)
            cnt = scan_chunk(1, cnt)
            return cnt

        cnt = lax.fori_loop(0, NSCAN // 2, scan_pair, 0)

        lovec = jnp.full((16,), lo * 32768, dtype=jnp.int32)
        for t in range(2 * CH // 16 + 1):
            ppack[pl.ds(cnt + 16 * t, 16)] = lovec

        def unpack_body(j, carry):
            val = ppack[pl.ds(16 * j, 16)]
            dvu = jnp.clip(jnp.right_shift(val, 15), 0, N - 1)
            svu = jnp.clip(jnp.bitwise_and(val, 32767), 0, N - 1)
            pdst[pl.ds(16 * j, 16)] = dvu
            psrc[pl.ds(16 * j, 16)] = svu
            return carry

        lax.fori_loop(0, PBUF // 16, unpack_body, 0)

        ntot = 2 * ((cnt + 2 * CH - 1) // (2 * CH))

        def issue_rows(g, b, sm):
            base = g * CH
            pltpu.async_copy(q_hbm.at[pdst.at[pl.ds(base, CH)]], qv.at[b], sm)
            pltpu.async_copy(k_hbm.at[psrc.at[pl.ds(base, CH)]], kv.at[b], sm)
            pltpu.async_copy(v_hbm.at[psrc.at[pl.ds(base, CH)]], vv.at[b], sm)

        def wait_rows(b, sm):
            pltpu.make_async_copy(q_hbm.at[pl.ds(0, CH)], qv.at[b], sm).wait()
            pltpu.make_async_copy(k_hbm.at[pl.ds(0, CH)], kv.at[b], sm).wait()
            pltpu.make_async_copy(v_hbm.at[pl.ds(0, CH)], vv.at[b], sm).wait()

        def compute_chunk(g, b):
            base = g * CH

            def edge_body(i, carry2):
                acc = qv[b, i, pl.ds(0, 16)] * kv[b, i, pl.ds(0, 16)]
                for j in range(1, 8):
                    acc = acc + (qv[b, i, pl.ds(16 * j, 16)]
                                 * kv[b, i, pl.ds(16 * j, 16)])
                for p in perms:
                    acc = acc + _permute16(acc, p)
                exvec = jnp.exp(acc * INV_SQRT)
                isreal = (base + i) < cnt
                exvec = jnp.where(isreal, exvec, 0.0)
                dstloc = pdst[pl.ds(base + i, 16)][0] - lo
                for j in range(8):
                    contrib = vv[b, i, pl.ds(16 * j, 16)] * exvec
                    plsc.addupdate(accv.at[dstloc, pl.ds(16 * j, 16)], contrib)
                exrow = jnp.where(lane == 0, exvec, 0.0)
                plsc.addupdate(denv.at[dstloc], exrow)
                return carry2

            lax.fori_loop(0, CH, edge_body, 0)

        @pl.when(ntot > 0)
        def _():
            issue_rows(0, 0, sem0)

        def pair_body(g2, carry):
            g0 = 2 * g2
            issue_rows(g0 + 1, 1, sem1)
            wait_rows(0, sem0)
            compute_chunk(g0, 0)

            @pl.when(g0 + 2 < ntot)
            def _():
                issue_rows(g0 + 2, 0, sem0)

            wait_rows(1, sem1)
            compute_chunk(g0 + 1, 1)
            return carry

        lax.fori_loop(0, ntot // 2, pair_body, 0)

        pltpu.sync_copy(accv, outp_hbm.at[pl.ds(lo, NT)])
        pltpu.sync_copy(denv, denp_hbm.at[pl.ds(lo, NT)])

    return body(q, k, v, src, dst)


def kernel(x, edge_index, Wq, bq, Wk, bk, Wv, bv, Ws, bs):
    wt = jnp.concatenate([Wq.T, Wk.T, Wv.T, Ws.T], axis=1)
    b = jnp.concatenate([bq, bk, bv, bs]).reshape(1, 512)
    proj = _proj(x, wt, b)
    q = proj[:, 0:128]
    k = proj[:, 128:256]
    v = proj[:, 256:384]
    skip = proj[:, 384:512]
    src = edge_index[0].astype(jnp.int32)
    dst = edge_index[1].astype(jnp.int32)
    outp, denp = _sc_edge(q, k, v, src, dst)
    return _combine(outp[:N], denp[:N], skip)


# speedup vs baseline: 7.3549x; 1.3520x over previous
import functools

import numpy as np

import jax
import jax.numpy as jnp
from jax import lax
from jax.experimental import pallas as pl
from jax.experimental.pallas import tpu as pltpu
from jax.experimental.pallas import tpu_sc as plsc

N = 10000
E = 320000
D = 128
NC = 2
NS = 16
NW = NC * NS
NPAD = 10240
NT = NPAD // NW
SCCH = 3200
CH = 40
PCAP = 12992
PBUF = PCAP + 4 * CH + 16
INV_SQRT = 1.0 / (128.0 ** 0.5)


BN = 400


def _proj_body(x_ref, wt_ref, b_ref, o_ref):
    o_ref[...] = (
        jnp.dot(x_ref[...], wt_ref[...], preferred_element_type=jnp.float32)
        + b_ref[...]
    )


def _proj(x, wt, b):
    return pl.pallas_call(
        _proj_body,
        grid=(N // BN,),
        in_specs=[
            pl.BlockSpec((BN, 128), lambda i: (i, 0)),
            pl.BlockSpec((128, 512), lambda i: (0, 0)),
            pl.BlockSpec((1, 512), lambda i: (0, 0)),
        ],
        out_specs=pl.BlockSpec((BN, 512), lambda i: (i, 0)),
        out_shape=jax.ShapeDtypeStruct((N, 512), jnp.float32),
    )(x, wt, b)


def _comb_body(p_ref, d_ref, s_ref, o_ref):
    den = d_ref[:, 0:1]
    o_ref[...] = p_ref[...] / (den + 1e-16) + s_ref[...]


def _combine(p, d, skip):
    return pl.pallas_call(
        _comb_body,
        grid=(N // BN,),
        in_specs=[
            pl.BlockSpec((BN, 128), lambda i: (i, 0)),
            pl.BlockSpec((BN, 16), lambda i: (i, 0)),
            pl.BlockSpec((BN, 128), lambda i: (i, 0)),
        ],
        out_specs=pl.BlockSpec((BN, 128), lambda i: (i, 0)),
        out_shape=jax.ShapeDtypeStruct((N, 128), jnp.float32),
    )(p, d, skip)


def _permute16(x, idx):
    return lax.gather(
        x,
        idx[:, None],
        dimension_numbers=lax.GatherDimensionNumbers(
            offset_dims=(), collapsed_slice_dims=(0,), start_index_map=(0,)
        ),
        slice_sizes=(1,),
        mode=lax.GatherScatterMode.PROMISE_IN_BOUNDS,
    )


def _sc_edge(q, k, v, src, dst):
    mesh = plsc.VectorSubcoreMesh(core_axis_name="c", subcore_axis_name="s")

    @functools.partial(
        pl.kernel,
        mesh=mesh,
        out_type=[
            jax.ShapeDtypeStruct((NPAD, D), jnp.float32),
            jax.ShapeDtypeStruct((NPAD, 16), jnp.float32),
        ],
        scratch_types=[
            pltpu.VMEM((2, SCCH), jnp.int32),
            pltpu.VMEM((2, SCCH), jnp.int32),
            pltpu.VMEM((PBUF,), jnp.int32),
            pltpu.VMEM((PBUF,), jnp.int32),
            pltpu.VMEM((PBUF,), jnp.int32),
            pltpu.VMEM((4, 16), jnp.int32),
            pltpu.VMEM((2, CH, D), jnp.float32),
            pltpu.VMEM((2, CH, D), jnp.float32),
            pltpu.VMEM((2, CH, D), jnp.float32),
            pltpu.VMEM((NT, D), jnp.float32),
            pltpu.VMEM((NT, 16), jnp.float32),
            pltpu.SemaphoreType.DMA,
            pltpu.SemaphoreType.DMA,
            pltpu.SemaphoreType.DMA,
            pltpu.SemaphoreType.DMA,
        ],
        compiler_params=pltpu.CompilerParams(use_tc_tiling_on_sc=False),
    )
    def body(q_hbm, k_hbm, v_hbm, src_hbm, dst_hbm,
             outp_hbm, denp_hbm,
             sidx, didx, psrc, pdst, ppack, bounce, qv, kv, vv, accv, denv,
             semA, semB, sem0, sem1):
        c = lax.axis_index("c")
        s = lax.axis_index("s")
        wid = c * NS + s
        lo = wid * NT

        lane = lax.iota(jnp.int32, 16)
        zero = jnp.zeros((16,), jnp.float32)
        perms = [jnp.bitwise_xor(lane, jnp.int32(sh)) for sh in (8, 4, 2, 1)]
        shift_perms = [jnp.bitwise_and(lane - sh, 15) for sh in (1, 2, 4, 8)]
        shift_gates = [jnp.where(lane >= sh, 1, 0) for sh in (1, 2, 4, 8)]
        jplus1 = lane + 1

        def zfill_body(i, carry):
            for j in range(8):
                accv[i, pl.ds(16 * j, 16)] = zero
            denv[i, :] = zero
            return carry

        lax.fori_loop(0, NT, zfill_body, 0)

        NSCAN = E // SCCH

        def issue_idx(g, b, sm):
            base = g * SCCH
            pltpu.async_copy(src_hbm.at[pl.ds(base, SCCH)], sidx.at[b], sm)
            pltpu.async_copy(dst_hbm.at[pl.ds(base, SCCH)], didx.at[b], sm)

        def wait_idx(b, sm):
            pltpu.make_async_copy(src_hbm.at[pl.ds(0, SCCH)], sidx.at[b], sm).wait()
            pltpu.make_async_copy(dst_hbm.at[pl.ds(0, SCCH)], didx.at[b], sm).wait()

        def scan_chunk(b, cnt):
            def vec_body(i4, cnt2):
                counts = []
                svals_l = []
                for u in range(4):
                    i = 4 * i4 + u
                    dv = didx[b, pl.ds(16 * i, 16)]
                    sv = sidx[b, pl.ds(16 * i, 16)]
                    m = (dv >= lo) & (dv < lo + NT)
                    mi = jnp.where(m, 1, 0)
                    incl = mi
                    for sp, sg in zip(shift_perms, shift_gates):
                        incl = incl + _permute16(incl, sp) * sg
                    bounce[u, :] = incl
                    counts.append(bounce[u, :][15])
                    pos = jnp.zeros((16,), jnp.int32)
                    for sh in (8, 4, 2, 1):
                        probe = _permute16(incl, pos + (sh - 1))
                        pos = pos + jnp.where(probe < jplus1, sh, 0)
                    vals = dv * 32768 + sv
                    svals_l.append(_permute16(vals, pos))
                for u in range(4):
                    off = jnp.minimum(cnt2, PCAP)
                    ppack[pl.ds(off, 16)] = svals_l[u]
                    cnt2 = off + counts[u]
                return cnt2

            return lax.fori_loop(0, SCCH // 64, vec_body, cnt)

        issue_idx(0, 0, semA)

        def scan_pair(g2, cnt):
            g0 = 2 * g2
            issue_idx(g0 + 1, 1, semB)
            wait_idx(0, semA)
            cnt = scan_chunk(0, cnt)

            @pl.when(g0 + 2 < NSCAN)
            def _():
                issue_idx(g0 + 2, 0, semA)

            wait_idx(1, semB)
            cnt = scan_chunk(1, cnt)
            return cnt

        cnt = lax.fori_loop(0, NSCAN // 2, scan_pair, 0)

        lovec = jnp.full((16,), lo * 32768, dtype=jnp.int32)
        for t in range(2 * CH // 16 + 1):
            ppack[pl.ds(cnt + 16 * t, 16)] = lovec

        def unpack_body(j, carry):
            val = ppack[pl.ds(16 * j, 16)]
            dvu = jnp.clip(jnp.right_shift(val, 15), 0, N - 1)
            svu = jnp.clip(jnp.bitwise_and(val, 32767), 0, N - 1)
            pdst[pl.ds(16 * j, 16)] = dvu
            psrc[pl.ds(16 * j, 16)] = svu
            return carry

        lax.fori_loop(0, PBUF // 16, unpack_body, 0)

        ntot = 2 * ((cnt + 2 * CH - 1) // (2 * CH))

        def issue_rows(g, b, sm):
            base = g * CH
            pltpu.async_copy(q_hbm.at[pdst.at[pl.ds(base, CH)]], qv.at[b], sm)
            pltpu.async_copy(k_hbm.at[psrc.at[pl.ds(base, CH)]], kv.at[b], sm)
            pltpu.async_copy(v_hbm.at[psrc.at[pl.ds(base, CH)]], vv.at[b], sm)

        def wait_rows(b, sm):
            pltpu.make_async_copy(q_hbm.at[pl.ds(0, CH)], qv.at[b], sm).wait()
            pltpu.make_async_copy(k_hbm.at[pl.ds(0, CH)], kv.at[b], sm).wait()
            pltpu.make_async_copy(v_hbm.at[pl.ds(0, CH)], vv.at[b], sm).wait()

        def compute_chunk(g, b):
            base = g * CH

            def edge_body(i2, carry2):
                for u in range(2):
                    i = 2 * i2 + u
                    acc = qv[b, i, pl.ds(0, 16)] * kv[b, i, pl.ds(0, 16)]
                    for j in range(1, 8):
                        acc = acc + (qv[b, i, pl.ds(16 * j, 16)]
                                     * kv[b, i, pl.ds(16 * j, 16)])
                    for p in perms:
                        acc = acc + _permute16(acc, p)
                    exvec = jnp.exp(acc * INV_SQRT)
                    isreal = (base + i) < cnt
                    exvec = jnp.where(isreal, exvec, 0.0)
                    dstloc = pdst[pl.ds(base + i, 16)][0] - lo
                    for j in range(8):
                        contrib = vv[b, i, pl.ds(16 * j, 16)] * exvec
                        plsc.addupdate(accv.at[dstloc, pl.ds(16 * j, 16)],
                                       contrib)
                    exrow = jnp.where(lane == 0, exvec, 0.0)
                    plsc.addupdate(denv.at[dstloc], exrow)
                return carry2

            lax.fori_loop(0, CH // 2, edge_body, 0)

        @pl.when(ntot > 0)
        def _():
            issue_rows(0, 0, sem0)

        def pair_body(g2, carry):
            g0 = 2 * g2
            issue_rows(g0 + 1, 1, sem1)
            wait_rows(0, sem0)
            compute_chunk(g0, 0)

            @pl.when(g0 + 2 < ntot)
            def _():
                issue_rows(g0 + 2, 0, sem0)

            wait_rows(1, sem1)
            compute_chunk(g0 + 1, 1)
            return carry

        lax.fori_loop(0, ntot // 2, pair_body, 0)

        pltpu.sync_copy(accv, outp_hbm.at[pl.ds(lo, NT)])
        pltpu.sync_copy(denv, denp_hbm.at[pl.ds(lo, NT)])

    return body(q, k, v, src, dst)


def kernel(x, edge_index, Wq, bq, Wk, bk, Wv, bv, Ws, bs):
    wt = jnp.concatenate([Wq.T, Wk.T, Wv.T, Ws.T], axis=1)
    b = jnp.concatenate([bq, bk, bv, bs]).reshape(1, 512)
    proj = _proj(x, wt, b)
    q = proj[:, 0:128]
    k = proj[:, 128:256]
    v = proj[:, 256:384]
    skip = proj[:, 384:512]
    src = edge_index[0].astype(jnp.int32)
    dst = edge_index[1].astype(jnp.int32)
    outp, denp = _sc_edge(q, k, v, src, dst)
    return _combine(outp[:N], denp[:N], skip)


# speedup vs baseline: 8.8894x; 1.2086x over previous
import functools

import numpy as np

import jax
import jax.numpy as jnp
from jax import lax
from jax.experimental import pallas as pl
from jax.experimental.pallas import tpu as pltpu
from jax.experimental.pallas import tpu_sc as plsc

N = 10000
E = 320000
D = 128
NC = 2
NS = 16
NW = NC * NS
NPAD = 10240
NT = NPAD // NW
SCCH = 3200
CH = 40
PCAP = 12992
PBUF = PCAP + 4 * CH + 16
INV_SQRT = 1.0 / (128.0 ** 0.5)


BN = 400


def _proj_body(x_ref, wt_ref, b_ref, o_ref):
    o_ref[...] = (
        jnp.dot(x_ref[...], wt_ref[...], preferred_element_type=jnp.float32)
        + b_ref[...]
    )


def _proj(x, wt, b):
    return pl.pallas_call(
        _proj_body,
        grid=(N // BN,),
        in_specs=[
            pl.BlockSpec((BN, 128), lambda i: (i, 0)),
            pl.BlockSpec((128, 512), lambda i: (0, 0)),
            pl.BlockSpec((1, 512), lambda i: (0, 0)),
        ],
        out_specs=pl.BlockSpec((BN, 512), lambda i: (i, 0)),
        out_shape=jax.ShapeDtypeStruct((N, 512), jnp.float32),
    )(x, wt, b)


def _comb_body(p_ref, d_ref, s_ref, o_ref):
    den = d_ref[:, 0:1]
    o_ref[...] = p_ref[...] / (den + 1e-16) + s_ref[...]


def _combine(p, d, skip):
    return pl.pallas_call(
        _comb_body,
        grid=(N // BN,),
        in_specs=[
            pl.BlockSpec((BN, 128), lambda i: (i, 0)),
            pl.BlockSpec((BN, 16), lambda i: (i, 0)),
            pl.BlockSpec((BN, 128), lambda i: (i, 0)),
        ],
        out_specs=pl.BlockSpec((BN, 128), lambda i: (i, 0)),
        out_shape=jax.ShapeDtypeStruct((N, 128), jnp.float32),
    )(p, d, skip)


def _permute16(x, idx):
    return lax.gather(
        x,
        idx[:, None],
        dimension_numbers=lax.GatherDimensionNumbers(
            offset_dims=(), collapsed_slice_dims=(0,), start_index_map=(0,)
        ),
        slice_sizes=(1,),
        mode=lax.GatherScatterMode.PROMISE_IN_BOUNDS,
    )


def _sc_edge(q, k, v, src, dst):
    mesh = plsc.VectorSubcoreMesh(core_axis_name="c", subcore_axis_name="s")

    @functools.partial(
        pl.kernel,
        mesh=mesh,
        out_type=[
            jax.ShapeDtypeStruct((NPAD, D), jnp.float32),
            jax.ShapeDtypeStruct((NPAD, 16), jnp.float32),
        ],
        scratch_types=[
            pltpu.VMEM((2, SCCH), jnp.int32),
            pltpu.VMEM((2, SCCH), jnp.int32),
            pltpu.VMEM((PBUF,), jnp.int32),
            pltpu.VMEM((PBUF,), jnp.int32),
            pltpu.VMEM((PBUF,), jnp.int32),
            pltpu.VMEM((4, 16), jnp.int32),
            pltpu.VMEM((2, CH, D), jnp.float32),
            pltpu.VMEM((2, CH, D), jnp.float32),
            pltpu.VMEM((2, CH, D), jnp.float32),
            pltpu.VMEM((NT, D), jnp.float32),
            pltpu.VMEM((NT, 16), jnp.float32),
            pltpu.SemaphoreType.DMA,
            pltpu.SemaphoreType.DMA,
            pltpu.SemaphoreType.DMA,
            pltpu.SemaphoreType.DMA,
        ],
        compiler_params=pltpu.CompilerParams(use_tc_tiling_on_sc=False),
    )
    def body(q_hbm, k_hbm, v_hbm, src_hbm, dst_hbm,
             outp_hbm, denp_hbm,
             sidx, didx, psrc, pdst, ppack, bounce, qv, kv, vv, accv, denv,
             semA, semB, sem0, sem1):
        c = lax.axis_index("c")
        s = lax.axis_index("s")
        wid = c * NS + s
        lo = wid * NT

        lane = lax.iota(jnp.int32, 16)
        zero = jnp.zeros((16,), jnp.float32)
        perms = [jnp.bitwise_xor(lane, jnp.int32(sh)) for sh in (8, 4, 2, 1)]
        shift_perms = [jnp.bitwise_and(lane - sh, 15) for sh in (1, 2, 4, 8)]
        shift_gates = [jnp.where(lane >= sh, 1, 0) for sh in (1, 2, 4, 8)]
        jplus1 = lane + 1

        def zfill_body(i, carry):
            for j in range(8):
                accv[i, pl.ds(16 * j, 16)] = zero
            denv[i, :] = zero
            return carry

        lax.fori_loop(0, NT, zfill_body, 0)

        NSCAN = E // SCCH

        def issue_idx(g, b, sm):
            base = g * SCCH
            pltpu.async_copy(src_hbm.at[pl.ds(base, SCCH)], sidx.at[b], sm)
            pltpu.async_copy(dst_hbm.at[pl.ds(base, SCCH)], didx.at[b], sm)

        def wait_idx(b, sm):
            pltpu.make_async_copy(src_hbm.at[pl.ds(0, SCCH)], sidx.at[b], sm).wait()
            pltpu.make_async_copy(dst_hbm.at[pl.ds(0, SCCH)], didx.at[b], sm).wait()

        def scan_chunk(b, cnt):
            def vec_body(i4, cnt2):
                counts = []
                svals_l = []
                for u in range(4):
                    i = 4 * i4 + u
                    dv = didx[b, pl.ds(16 * i, 16)]
                    sv = sidx[b, pl.ds(16 * i, 16)]
                    m = (dv >= lo) & (dv < lo + NT)
                    mi = jnp.where(m, 1, 0)
                    incl = mi
                    for sp, sg in zip(shift_perms, shift_gates):
                        incl = incl + _permute16(incl, sp) * sg
                    bounce[u, :] = incl
                    counts.append(bounce[u, :][15])
                    pos = jnp.zeros((16,), jnp.int32)
                    for sh in (8, 4, 2, 1):
                        probe = _permute16(incl, pos + (sh - 1))
                        pos = pos + jnp.where(probe < jplus1, sh, 0)
                    vals = dv * 32768 + sv
                    svals_l.append(_permute16(vals, pos))
                for u in range(4):
                    off = jnp.minimum(cnt2, PCAP)
                    ppack[pl.ds(off, 16)] = svals_l[u]
                    cnt2 = off + counts[u]
                return cnt2

            return lax.fori_loop(0, SCCH // 64, vec_body, cnt)

        issue_idx(0, 0, semA)

        def scan_pair(g2, cnt):
            g0 = 2 * g2
            issue_idx(g0 + 1, 1, semB)
            wait_idx(0, semA)
            cnt = scan_chunk(0, cnt)

            @pl.when(g0 + 2 < NSCAN)
            def _():
                issue_idx(g0 + 2, 0, semA)

            wait_idx(1, semB)
            cnt = scan_chunk(1, cnt)
            return cnt

        cnt = lax.fori_loop(0, NSCAN // 2, scan_pair, 0)

        lovec = jnp.full((16,), lo * 32768, dtype=jnp.int32)
        for t in range(2 * CH // 16 + 1):
            ppack[pl.ds(cnt + 16 * t, 16)] = lovec

        def unpack_body(j, carry):
            val = ppack[pl.ds(16 * j, 16)]
            dvu = jnp.clip(jnp.right_shift(val, 15), 0, N - 1)
            svu = jnp.clip(jnp.bitwise_and(val, 32767), 0, N - 1)
            pdst[pl.ds(16 * j, 16)] = dvu
            psrc[pl.ds(16 * j, 16)] = svu
            return carry

        lax.fori_loop(0, PBUF // 16, unpack_body, 0)

        ntot = 2 * ((cnt + 2 * CH - 1) // (2 * CH))

        def issue_rows(g, b, sm):
            base = g * CH
            pltpu.async_copy(q_hbm.at[pdst.at[pl.ds(base, CH)]], qv.at[b], sm)
            pltpu.async_copy(k_hbm.at[psrc.at[pl.ds(base, CH)]], kv.at[b], sm)
            pltpu.async_copy(v_hbm.at[psrc.at[pl.ds(base, CH)]], vv.at[b], sm)

        def wait_rows(b, sm):
            pltpu.make_async_copy(q_hbm.at[pl.ds(0, CH)], qv.at[b], sm).wait()
            pltpu.make_async_copy(k_hbm.at[pl.ds(0, CH)], kv.at[b], sm).wait()
            pltpu.make_async_copy(v_hbm.at[pl.ds(0, CH)], vv.at[b], sm).wait()

        def compute_chunk(g, b):
            base = g * CH

            def edge_body(i4, carry2):
                dvec = pdst[pl.ds(base + 4 * i4, 16)]
                exvecs = []
                dstlocs = []
                for u in range(4):
                    i = 4 * i4 + u
                    acc = qv[b, i, pl.ds(0, 16)] * kv[b, i, pl.ds(0, 16)]
                    for j in range(1, 8):
                        acc = acc + (qv[b, i, pl.ds(16 * j, 16)]
                                     * kv[b, i, pl.ds(16 * j, 16)])
                    for p in perms:
                        acc = acc + _permute16(acc, p)
                    exvec = jnp.exp(acc * INV_SQRT)
                    isreal = (base + i) < cnt
                    exvecs.append(jnp.where(isreal, exvec, 0.0))
                    dstlocs.append(dvec[u] - lo)
                for u in range(4):
                    i = 4 * i4 + u
                    exvec = exvecs[u]
                    dstloc = dstlocs[u]
                    for j in range(8):
                        contrib = vv[b, i, pl.ds(16 * j, 16)] * exvec
                        plsc.addupdate(accv.at[dstloc, pl.ds(16 * j, 16)],
                                       contrib)
                    exrow = jnp.where(lane == 0, exvec, 0.0)
                    plsc.addupdate(denv.at[dstloc], exrow)
                return carry2

            lax.fori_loop(0, CH // 4, edge_body, 0)

        @pl.when(ntot > 0)
        def _():
            issue_rows(0, 0, sem0)

        def pair_body(g2, carry):
            g0 = 2 * g2
            issue_rows(g0 + 1, 1, sem1)
            wait_rows(0, sem0)
            compute_chunk(g0, 0)

            @pl.when(g0 + 2 < ntot)
            def _():
                issue_rows(g0 + 2, 0, sem0)

            wait_rows(1, sem1)
            compute_chunk(g0 + 1, 1)
            return carry

        lax.fori_loop(0, ntot // 2, pair_body, 0)

        pltpu.sync_copy(accv, outp_hbm.at[pl.ds(lo, NT)])
        pltpu.sync_copy(denv, denp_hbm.at[pl.ds(lo, NT)])

    return body(q, k, v, src, dst)


def kernel(x, edge_index, Wq, bq, Wk, bk, Wv, bv, Ws, bs):
    wt = jnp.concatenate([Wq.T, Wk.T, Wv.T, Ws.T], axis=1)
    b = jnp.concatenate([bq, bk, bv, bs]).reshape(1, 512)
    proj = _proj(x, wt, b)
    q = proj[:, 0:128]
    k = proj[:, 128:256]
    v = proj[:, 256:384]
    skip = proj[:, 384:512]
    src = edge_index[0].astype(jnp.int32)
    dst = edge_index[1].astype(jnp.int32)
    outp, denp = _sc_edge(q, k, v, src, dst)
    return _combine(outp[:N], denp[:N], skip)


# speedup vs baseline: 9.1880x; 1.0336x over previous
import functools

import numpy as np

import jax
import jax.numpy as jnp
from jax import lax
from jax.experimental import pallas as pl
from jax.experimental.pallas import tpu as pltpu
from jax.experimental.pallas import tpu_sc as plsc

N = 10000
E = 320000
D = 128
NC = 2
NS = 16
NW = NC * NS
NPAD = 10240
NT = NPAD // NW
SCCH = 3200
CH = 40
PCAP = 12992
PBUF = PCAP + 4 * CH + 16
INV_SQRT = 1.0 / (128.0 ** 0.5)


BN = 400


def _proj_body(x_ref, wt_ref, b_ref, o_ref):
    o_ref[...] = (
        jnp.dot(x_ref[...], wt_ref[...], preferred_element_type=jnp.float32)
        + b_ref[...]
    )


def _proj(x, wt, b):
    return pl.pallas_call(
        _proj_body,
        grid=(N // BN,),
        in_specs=[
            pl.BlockSpec((BN, 128), lambda i: (i, 0)),
            pl.BlockSpec((128, 512), lambda i: (0, 0)),
            pl.BlockSpec((1, 512), lambda i: (0, 0)),
        ],
        out_specs=pl.BlockSpec((BN, 512), lambda i: (i, 0)),
        out_shape=jax.ShapeDtypeStruct((N, 512), jnp.float32),
    )(x, wt, b)


def _comb_body(p_ref, d_ref, s_ref, o_ref):
    den = d_ref[:, 0:1]
    o_ref[...] = p_ref[...] / (den + 1e-16) + s_ref[...]


def _combine(p, d, skip):
    return pl.pallas_call(
        _comb_body,
        grid=(N // BN,),
        in_specs=[
            pl.BlockSpec((BN, 128), lambda i: (i, 0)),
            pl.BlockSpec((BN, 16), lambda i: (i, 0)),
            pl.BlockSpec((BN, 128), lambda i: (i, 0)),
        ],
        out_specs=pl.BlockSpec((BN, 128), lambda i: (i, 0)),
        out_shape=jax.ShapeDtypeStruct((N, 128), jnp.float32),
    )(p, d, skip)


def _permute16(x, idx):
    return lax.gather(
        x,
        idx[:, None],
        dimension_numbers=lax.GatherDimensionNumbers(
            offset_dims=(), collapsed_slice_dims=(0,), start_index_map=(0,)
        ),
        slice_sizes=(1,),
        mode=lax.GatherScatterMode.PROMISE_IN_BOUNDS,
    )


def _sc_edge(q, k, v, src, dst):
    mesh = plsc.VectorSubcoreMesh(core_axis_name="c", subcore_axis_name="s")

    @functools.partial(
        pl.kernel,
        mesh=mesh,
        out_type=[
            jax.ShapeDtypeStruct((NPAD, D), jnp.float32),
            jax.ShapeDtypeStruct((NPAD, 16), jnp.float32),
        ],
        scratch_types=[
            pltpu.VMEM((2, SCCH), jnp.int32),
            pltpu.VMEM((2, SCCH), jnp.int32),
            pltpu.VMEM((PBUF,), jnp.int32),
            pltpu.VMEM((PBUF,), jnp.int32),
            pltpu.VMEM((PBUF,), jnp.int32),
            pltpu.VMEM((8, 16), jnp.int32),
            pltpu.VMEM((2, CH, D), jnp.float32),
            pltpu.VMEM((2, CH, D), jnp.float32),
            pltpu.VMEM((2, CH, D), jnp.float32),
            pltpu.VMEM((NT, D), jnp.float32),
            pltpu.VMEM((NT, 16), jnp.float32),
            pltpu.SemaphoreType.DMA,
            pltpu.SemaphoreType.DMA,
            pltpu.SemaphoreType.DMA,
            pltpu.SemaphoreType.DMA,
        ],
        compiler_params=pltpu.CompilerParams(use_tc_tiling_on_sc=False),
    )
    def body(q_hbm, k_hbm, v_hbm, src_hbm, dst_hbm,
             outp_hbm, denp_hbm,
             sidx, didx, psrc, pdst, ppack, bounce, qv, kv, vv, accv, denv,
             semA, semB, sem0, sem1):
        c = lax.axis_index("c")
        s = lax.axis_index("s")
        wid = c * NS + s
        lo = wid * NT

        lane = lax.iota(jnp.int32, 16)
        zero = jnp.zeros((16,), jnp.float32)
        perms = [jnp.bitwise_xor(lane, jnp.int32(sh)) for sh in (8, 4, 2, 1)]
        shift_perms = [jnp.bitwise_and(lane - sh, 15) for sh in (1, 2, 4, 8)]
        shift_gates = [jnp.where(lane >= sh, 1, 0) for sh in (1, 2, 4, 8)]
        jplus1 = lane + 1

        def zfill_body(i, carry):
            for j in range(8):
                accv[i, pl.ds(16 * j, 16)] = zero
            denv[i, :] = zero
            return carry

        lax.fori_loop(0, NT, zfill_body, 0)

        NSCAN = E // SCCH

        def issue_idx(g, b, sm):
            base = g * SCCH
            pltpu.async_copy(src_hbm.at[pl.ds(base, SCCH)], sidx.at[b], sm)
            pltpu.async_copy(dst_hbm.at[pl.ds(base, SCCH)], didx.at[b], sm)

        def wait_idx(b, sm):
            pltpu.make_async_copy(src_hbm.at[pl.ds(0, SCCH)], sidx.at[b], sm).wait()
            pltpu.make_async_copy(dst_hbm.at[pl.ds(0, SCCH)], didx.at[b], sm).wait()

        def scan_chunk(b, cnt):
            def vec_body(i4, cnt2):
                counts = []
                svals_l = []
                for u in range(8):
                    i = 8 * i4 + u
                    dv = didx[b, pl.ds(16 * i, 16)]
                    sv = sidx[b, pl.ds(16 * i, 16)]
                    m = (dv >= lo) & (dv < lo + NT)
                    mi = jnp.where(m, 1, 0)
                    incl = mi
                    for sp, sg in zip(shift_perms, shift_gates):
                        incl = incl + _permute16(incl, sp) * sg
                    bounce[u, :] = incl
                    counts.append(bounce[u, :][15])
                    pos = jnp.zeros((16,), jnp.int32)
                    for sh in (8, 4, 2, 1):
                        probe = _permute16(incl, pos + (sh - 1))
                        pos = pos + jnp.where(probe < jplus1, sh, 0)
                    vals = dv * 32768 + sv
                    svals_l.append(_permute16(vals, pos))
                for u in range(8):
                    off = jnp.minimum(cnt2, PCAP)
                    ppack[pl.ds(off, 16)] = svals_l[u]
                    cnt2 = off + counts[u]
                return cnt2

            return lax.fori_loop(0, SCCH // 128, vec_body, cnt)

        issue_idx(0, 0, semA)

        def scan_pair(g2, cnt):
            g0 = 2 * g2
            issue_idx(g0 + 1, 1, semB)
            wait_idx(0, semA)
            cnt = scan_chunk(0, cnt)

            @pl.when(g0 + 2 < NSCAN)
            def _():
                issue_idx(g0 + 2, 0, semA)

            wait_idx(1, semB)
            cnt = scan_chunk(1, cnt)
            return cnt

        cnt = lax.fori_loop(0, NSCAN // 2, scan_pair, 0)

        lovec = jnp.full((16,), lo * 32768, dtype=jnp.int32)
        for t in range(2 * CH // 16 + 1):
            ppack[pl.ds(cnt + 16 * t, 16)] = lovec

        def unpack_body(j, carry):
            val = ppack[pl.ds(16 * j, 16)]
            dvu = jnp.clip(jnp.right_shift(val, 15), 0, N - 1)
            svu = jnp.clip(jnp.bitwise_and(val, 32767), 0, N - 1)
            pdst[pl.ds(16 * j, 16)] = dvu
            psrc[pl.ds(16 * j, 16)] = svu
            return carry

        lax.fori_loop(0, PBUF // 16, unpack_body, 0)

        ntot = 2 * ((cnt + 2 * CH - 1) // (2 * CH))

        def issue_rows(g, b, sm):
            base = g * CH
            pltpu.async_copy(q_hbm.at[pdst.at[pl.ds(base, CH)]], qv.at[b], sm)
            pltpu.async_copy(k_hbm.at[psrc.at[pl.ds(base, CH)]], kv.at[b], sm)
            pltpu.async_copy(v_hbm.at[psrc.at[pl.ds(base, CH)]], vv.at[b], sm)

        def wait_rows(b, sm):
            pltpu.make_async_copy(q_hbm.at[pl.ds(0, CH)], qv.at[b], sm).wait()
            pltpu.make_async_copy(k_hbm.at[pl.ds(0, CH)], kv.at[b], sm).wait()
            pltpu.make_async_copy(v_hbm.at[pl.ds(0, CH)], vv.at[b], sm).wait()

        def compute_chunk(g, b):
            base = g * CH

            def edge_body(i4, carry2):
                dvec = pdst[pl.ds(base + 4 * i4, 16)]
                exvecs = []
                dstlocs = []
                for u in range(4):
                    i = 4 * i4 + u
                    acc = qv[b, i, pl.ds(0, 16)] * kv[b, i, pl.ds(0, 16)]
                    for j in range(1, 8):
                        acc = acc + (qv[b, i, pl.ds(16 * j, 16)]
                                     * kv[b, i, pl.ds(16 * j, 16)])
                    for p in perms:
                        acc = acc + _permute16(acc, p)
                    exvec = jnp.exp(acc * INV_SQRT)
                    isreal = (base + i) < cnt
                    exvecs.append(jnp.where(isreal, exvec, 0.0))
                    dstlocs.append(dvec[u] - lo)
                for u in range(4):
                    i = 4 * i4 + u
                    exvec = exvecs[u]
                    dstloc = dstlocs[u]
                    for j in range(8):
                        contrib = vv[b, i, pl.ds(16 * j, 16)] * exvec
                        plsc.addupdate(accv.at[dstloc, pl.ds(16 * j, 16)],
                                       contrib)
                    exrow = jnp.where(lane == 0, exvec, 0.0)
                    plsc.addupdate(denv.at[dstloc], exrow)
                return carry2

            lax.fori_loop(0, CH // 4, edge_body, 0)

        @pl.when(ntot > 0)
        def _():
            issue_rows(0, 0, sem0)

        def pair_body(g2, carry):
            g0 = 2 * g2
            issue_rows(g0 + 1, 1, sem1)
            wait_rows(0, sem0)
            compute_chunk(g0, 0)

            @pl.when(g0 + 2 < ntot)
            def _():
                issue_rows(g0 + 2, 0, sem0)

            wait_rows(1, sem1)
            compute_chunk(g0 + 1, 1)
            return carry

        lax.fori_loop(0, ntot // 2, pair_body, 0)

        pltpu.sync_copy(accv, outp_hbm.at[pl.ds(lo, NT)])
        pltpu.sync_copy(denv, denp_hbm.at[pl.ds(lo, NT)])

    return body(q, k, v, src, dst)


def kernel(x, edge_index, Wq, bq, Wk, bk, Wv, bv, Ws, bs):
    wt = jnp.concatenate([Wq.T, Wk.T, Wv.T, Ws.T], axis=1)
    b = jnp.concatenate([bq, bk, bv, bs]).reshape(1, 512)
    proj = _proj(x, wt, b)
    q = proj[:, 0:128]
    k = proj[:, 128:256]
    v = proj[:, 256:384]
    skip = proj[:, 384:512]
    src = edge_index[0].astype(jnp.int32)
    dst = edge_index[1].astype(jnp.int32)
    outp, denp = _sc_edge(q, k, v, src, dst)
    return _combine(outp[:N], denp[:N], skip)
